# baseline (speedup 1.0000x reference)
import jax
import jax.numpy as jnp
from jax.experimental import pallas as pl

_PACK = 8
_BLOCK = 2048


def _fused_edge_kernel(f_ref, wq_ref, wr_ref, p_ref, fout_ref):
    f = f_ref[...]
    p_ref[...] = jnp.dot(f, wq_ref[...], preferred_element_type=jnp.float32)
    fout_ref[...] = jnp.dot(f, wr_ref[...], preferred_element_type=jnp.float32)


def _output_kernel(a_ref, x_ref, w_ref, b_ref, o_ref):
    support = jnp.dot(x_ref[...], w_ref[...], preferred_element_type=jnp.float32)
    o_ref[...] = (
        jnp.dot(a_ref[...], support, preferred_element_type=jnp.float32)
        + b_ref[...]
    )


def kernel(input, adj, F, weight, weight_q, weight_r, bias):
    n, in_f = input.shape
    out_f = weight.shape[1]
    nn, edge_f = F.shape
    edge_out = weight_r.shape[1]
    lanes = _PACK * edge_f

    f8 = F.reshape(nn // _PACK, lanes)

    wq_big = jax.scipy.linalg.block_diag(*([weight_q] * _PACK))
    wr_big = jax.scipy.linalg.block_diag(*([weight_r] * _PACK))

    rows8 = nn // _PACK
    grid = rows8 // _BLOCK

    p8, fout8 = pl.pallas_call(
        _fused_edge_kernel,
        grid=(grid,),
        in_specs=[
            pl.BlockSpec((_BLOCK, lanes), lambda i: (i, 0)),
            pl.BlockSpec((lanes, _PACK), lambda i: (0, 0)),
            pl.BlockSpec((lanes, _PACK * edge_out), lambda i: (0, 0)),
        ],
        out_specs=[
            pl.BlockSpec((_BLOCK, _PACK), lambda i: (i, 0)),
            pl.BlockSpec((_BLOCK, _PACK * edge_out), lambda i: (i, 0)),
        ],
        out_shape=[
            jax.ShapeDtypeStruct((rows8, _PACK), jnp.float32),
            jax.ShapeDtypeStruct((rows8, _PACK * edge_out), jnp.float32),
        ],
    )(f8, wq_big, wr_big)

    A = p8.reshape(n, n)
    F_out = fout8.reshape(nn, edge_out)

    output = pl.pallas_call(
        _output_kernel,
        in_specs=[
            pl.BlockSpec((n, n), lambda: (0, 0)),
            pl.BlockSpec((n, in_f), lambda: (0, 0)),
            pl.BlockSpec((in_f, out_f), lambda: (0, 0)),
            pl.BlockSpec((1, out_f), lambda: (0, 0)),
        ],
        out_specs=pl.BlockSpec((n, out_f), lambda: (0, 0)),
        out_shape=jax.ShapeDtypeStruct((n, out_f), jnp.float32),
    )(A, input, weight, bias.reshape(1, out_f))

    return (output, F_out)


# speedup vs baseline: 19.7601x; 19.7601x over previous
import jax
import jax.numpy as jnp
from jax.experimental import pallas as pl
from jax.experimental.pallas import tpu as pltpu

_BLOCKN = 65536


def _make_kernel(n, nn, edge_f, edge_out, in_f, out_f, nsteps, rows):
    def fused(ftb_ref, wqt_ref, wrt_ref, x_ref, w_ref, b_ref,
              fout_ref, out_ref, a_scr, sup_scr):
        i = pl.program_id(0)

        @pl.when(i == 0)
        def _():
            sup_scr[...] = jnp.dot(x_ref[...], w_ref[...],
                                   preferred_element_type=jnp.float32)

        ftb = ftb_ref[...]
        fout_ref[...] = jnp.dot(wrt_ref[...], ftb,
                                preferred_element_type=jnp.float32)
        arow = jnp.dot(wqt_ref[...], ftb,
                       preferred_element_type=jnp.float32)
        a_scr[pl.ds(i * rows, rows), :] = arow.reshape(rows, n)

        @pl.when(i == nsteps - 1)
        def _():
            out_ref[...] = (
                jnp.dot(a_scr[...], sup_scr[...],
                        preferred_element_type=jnp.float32)
                + b_ref[...]
            )

    return fused


def kernel(input, adj, F, weight, weight_q, weight_r, bias):
    n, in_f = input.shape
    out_f = weight.shape[1]
    nn, edge_f = F.shape
    edge_out = weight_r.shape[1]

    ft = F.T
    wqt = weight_q.T
    wrt = weight_r.T

    nsteps = nn // _BLOCKN
    rows = _BLOCKN // n

    fused = _make_kernel(n, nn, edge_f, edge_out, in_f, out_f, nsteps, rows)

    fout_t, output = pl.pallas_call(
        fused,
        grid=(nsteps,),
        in_specs=[
            pl.BlockSpec((edge_f, _BLOCKN), lambda i: (0, i)),
            pl.BlockSpec((1, edge_f), lambda i: (0, 0)),
            pl.BlockSpec((edge_out, edge_f), lambda i: (0, 0)),
            pl.BlockSpec((n, in_f), lambda i: (0, 0)),
            pl.BlockSpec((in_f, out_f), lambda i: (0, 0)),
            pl.BlockSpec((1, out_f), lambda i: (0, 0)),
        ],
        out_specs=[
            pl.BlockSpec((edge_out, _BLOCKN), lambda i: (0, i)),
            pl.BlockSpec((n, out_f), lambda i: (0, 0)),
        ],
        out_shape=[
            jax.ShapeDtypeStruct((edge_out, nn), jnp.float32),
            jax.ShapeDtypeStruct((n, out_f), jnp.float32),
        ],
        scratch_shapes=[
            pltpu.VMEM((n, n), jnp.float32),
            pltpu.VMEM((n, out_f), jnp.float32),
        ],
    )(ft, wqt, wrt, input, weight, bias.reshape(1, out_f))

    return (output, fout_t.T)


# speedup vs baseline: 20.4582x; 1.0353x over previous
import jax
import jax.numpy as jnp
from jax.experimental import pallas as pl
from jax.experimental.pallas import tpu as pltpu

_BLOCKN = 131072


def _make_kernel(n, nn, edge_f, edge_out, in_f, out_f, nsteps, rows):
    def fused(ftb_ref, wqt_ref, wrt_ref, x_ref, w_ref, b_ref,
              fout_ref, out_ref, a_scr, sup_scr):
        i = pl.program_id(0)

        @pl.when(i == 0)
        def _():
            sup_scr[...] = jnp.dot(x_ref[...], w_ref[...],
                                   preferred_element_type=jnp.float32)

        ftb = ftb_ref[...]
        fout_ref[...] = jnp.dot(wrt_ref[...], ftb,
                                preferred_element_type=jnp.float32)
        arow = jnp.dot(wqt_ref[...], ftb,
                       preferred_element_type=jnp.float32)
        a_scr[pl.ds(i * rows, rows), :] = arow.reshape(rows, n)

        @pl.when(i == nsteps - 1)
        def _():
            out_ref[...] = (
                jnp.dot(a_scr[...], sup_scr[...],
                        preferred_element_type=jnp.float32)
                + b_ref[...]
            )

    return fused


def kernel(input, adj, F, weight, weight_q, weight_r, bias):
    n, in_f = input.shape
    out_f = weight.shape[1]
    nn, edge_f = F.shape
    edge_out = weight_r.shape[1]

    ft = F.T
    wqt = weight_q.T
    wrt = weight_r.T

    nsteps = nn // _BLOCKN
    rows = _BLOCKN // n

    fused = _make_kernel(n, nn, edge_f, edge_out, in_f, out_f, nsteps, rows)

    fout_t, output = pl.pallas_call(
        fused,
        grid=(nsteps,),
        in_specs=[
            pl.BlockSpec((edge_f, _BLOCKN), lambda i: (0, i)),
            pl.BlockSpec((1, edge_f), lambda i: (0, 0)),
            pl.BlockSpec((edge_out, edge_f), lambda i: (0, 0)),
            pl.BlockSpec((n, in_f), lambda i: (0, 0)),
            pl.BlockSpec((in_f, out_f), lambda i: (0, 0)),
            pl.BlockSpec((1, out_f), lambda i: (0, 0)),
        ],
        out_specs=[
            pl.BlockSpec((edge_out, _BLOCKN), lambda i: (0, i)),
            pl.BlockSpec((n, out_f), lambda i: (0, 0)),
        ],
        out_shape=[
            jax.ShapeDtypeStruct((edge_out, nn), jnp.float32),
            jax.ShapeDtypeStruct((n, out_f), jnp.float32),
        ],
        scratch_shapes=[
            pltpu.VMEM((n, n), jnp.float32),
            pltpu.VMEM((n, out_f), jnp.float32),
        ],
    )(ft, wqt, wrt, input, weight, bias.reshape(1, out_f))

    return (output, fout_t.T)
